# baseline (speedup 1.0000x reference)
import jax
import jax.numpy as jnp
from jax.experimental import pallas as pl
from jax.experimental.pallas import tpu as pltpu

_B, _D, _K, _HW = 8, 64, 512, 256
_BPS = 4
_W = _BPS * _HW
_NSTEP = _B // _BPS


def _vq_body(x_ref, et_ref, enc_ref, q_ref, loss_ref, perp_ref, cnt_ref,
             scr_ref):
    b = pl.program_id(0)
    x = jnp.concatenate([x_ref[i] for i in range(_BPS)], axis=1)
    Et = et_ref[...]

    b1 = Et.astype(jnp.bfloat16)
    r1 = Et - b1.astype(jnp.float32)
    b2 = r1.astype(jnp.bfloat16)
    r2 = r1 - b2.astype(jnp.float32)
    b3 = r2.astype(jnp.bfloat16)
    x1 = x.astype(jnp.bfloat16)
    x2 = (x - x1.astype(jnp.float32)).astype(jnp.bfloat16)

    def mm(lhs, rhs):
        return jax.lax.dot_general(lhs, rhs, (((1,), (0,)), ((), ())),
                                   preferred_element_type=jnp.float32)

    dots = mm(b1, x1) + (mm(b1, x2) + mm(b2, x1))
    e2 = jnp.sum(Et * Et, axis=1, keepdims=True)
    s = e2 - 2.0 * dots

    kio = jax.lax.broadcasted_iota(jnp.int32, (_K, _W), 0)
    m1 = jnp.min(s, axis=0, keepdims=True)
    eq1 = s == m1
    c1 = jnp.min(jnp.where(eq1, kio, _K), axis=0, keepdims=True)
    s2 = jnp.where(eq1, jnp.inf, s)
    m2 = jnp.min(s2, axis=0, keepdims=True)
    c2 = jnp.min(jnp.where(s2 == m2, kio, _K), axis=0, keepdims=True)

    b123 = jnp.concatenate([b1, b2, b3], axis=1)
    oh12 = jnp.concatenate([(kio == c1), (kio == c2)],
                           axis=1).astype(jnp.bfloat16)
    g = jax.lax.dot_general(b123, oh12, (((0,), (0,)), ((), ())),
                            preferred_element_type=jnp.float32)
    e12 = (g[:_D] + g[_D:2 * _D]) + g[2 * _D:]

    xx = jnp.concatenate([x, x], axis=1)
    df = e12 - xx
    scr_ref[...] = df * df

    def dstep(d, acc):
        return acc + scr_ref[pl.ds(d, 1), :]

    acc = jax.lax.fori_loop(0, _D, dstep, jnp.zeros((1, 2 * _W), jnp.float32))
    dist = jnp.sqrt(acc)
    d1, d2 = dist[:, :_W], dist[:, _W:]
    a1, a2 = acc[:, :_W], acc[:, _W:]

    win2 = (d2 < d1) | ((d2 == d1) & (c2 < c1))
    idx = jnp.where(win2, c2, c1)
    lsq = jnp.where(win2, a2, a1)

    onehot = (kio == idx).astype(jnp.float32)
    ew = jnp.where(win2, e12[:, _W:], e12[:, :_W])
    qst = x + (ew - x)
    for i in range(_BPS):
        enc_ref[i] = onehot[:, i * _HW:(i + 1) * _HW]
        q_ref[i] = qst[:, i * _HW:(i + 1) * _HW]
    contrib = jnp.sum(lsq).reshape(1, 1)

    @pl.when(b == 0)
    def _():
        cnt_ref[...] = onehot
        loss_ref[...] = contrib

    @pl.when(b > 0)
    def _():
        cnt_ref[...] += onehot
        loss_ref[...] += contrib

    @pl.when(b == _NSTEP - 1)
    def _():
        counts = jnp.sum(cnt_ref[...], axis=1, keepdims=True)
        p = counts * (1.0 / (_B * _HW))
        ent = -jnp.sum(p * jnp.log(p + 1e-10))
        perp_ref[...] = jnp.exp(ent).reshape(1, 1) * (1.0 / _K)
        loss_ref[...] = loss_ref[...] * (1.0 / (_B * _D * _HW))


def kernel(input, embedding):
    B, D, H, W = input.shape
    K = embedding.shape[1]
    x3 = input.reshape(B, D, H * W)
    Et = jnp.transpose(embedding[:, :, 0])
    enc, q, loss, perp = pl.pallas_call(
        _vq_body,
        grid=(_NSTEP,),
        in_specs=[
            pl.BlockSpec((_BPS, _D, _HW), lambda b: (b, 0, 0)),
            pl.BlockSpec((_K, _D), lambda b: (0, 0)),
        ],
        out_specs=[
            pl.BlockSpec((_BPS, _K, _HW), lambda b: (b, 0, 0)),
            pl.BlockSpec((_BPS, _D, _HW), lambda b: (b, 0, 0)),
            pl.BlockSpec((1, 1), lambda b: (0, 0)),
            pl.BlockSpec((1, 1), lambda b: (0, 0)),
        ],
        out_shape=[
            jax.ShapeDtypeStruct((_B, _K, _HW), jnp.float32),
            jax.ShapeDtypeStruct((_B, _D, _HW), jnp.float32),
            jax.ShapeDtypeStruct((1, 1), jnp.float32),
            jax.ShapeDtypeStruct((1, 1), jnp.float32),
        ],
        scratch_shapes=[
            pltpu.VMEM((_K, _W), jnp.float32),
            pltpu.VMEM((_D, 2 * _W), jnp.float32),
        ],
    )(x3, Et)
    quantized_out = q.reshape(B, D, H, W)
    encoding_out = enc.reshape(B, K, H, W)
    commitment_loss = loss[0, 0]
    perplexity = perp.reshape(1)
    return (quantized_out, encoding_out, commitment_loss, perplexity)


# speedup vs baseline: 1.0227x; 1.0227x over previous
import jax
import jax.numpy as jnp
from jax.experimental import pallas as pl
from jax.experimental.pallas import tpu as pltpu

_B, _D, _K, _HW = 8, 64, 512, 256
_BPS = 8
_W = _BPS * _HW
_NSTEP = _B // _BPS


def _vq_body(x_ref, e_ref, enc_ref, q_ref, loss_ref, perp_ref, cnt_ref,
             scr_ref):
    b = pl.program_id(0)
    x = jnp.concatenate([x_ref[i] for i in range(_BPS)], axis=1)
    E = e_ref[...]

    b1 = E.astype(jnp.bfloat16)
    r1 = E - b1.astype(jnp.float32)
    b2 = r1.astype(jnp.bfloat16)
    r2 = r1 - b2.astype(jnp.float32)
    b3 = r2.astype(jnp.bfloat16)
    x1 = x.astype(jnp.bfloat16)
    x2 = (x - x1.astype(jnp.float32)).astype(jnp.bfloat16)

    def mm(lhs, rhs):
        return jax.lax.dot_general(lhs, rhs, (((0,), (0,)), ((), ())),
                                   preferred_element_type=jnp.float32)

    dots = mm(b1, x1) + (mm(b1, x2) + mm(b2, x1))
    ones = jnp.ones((_D, 1), jnp.float32)
    e2 = jax.lax.dot_general(E * E, ones, (((0,), (0,)), ((), ())),
                             precision=jax.lax.Precision.HIGHEST,
                             preferred_element_type=jnp.float32)
    s = e2 - 2.0 * dots

    kio = jax.lax.broadcasted_iota(jnp.int32, (_K, _W), 0)
    m1 = jnp.min(s, axis=0, keepdims=True)
    eq1 = s == m1
    c1 = jnp.min(jnp.where(eq1, kio, _K), axis=0, keepdims=True)
    s2 = jnp.where(eq1, jnp.inf, s)
    m2 = jnp.min(s2, axis=0, keepdims=True)
    c2 = jnp.min(jnp.where(s2 == m2, kio, _K), axis=0, keepdims=True)

    b123 = jnp.concatenate([b1, b2, b3], axis=0)
    oh12 = jnp.concatenate([(kio == c1), (kio == c2)],
                           axis=1).astype(jnp.bfloat16)
    g = jax.lax.dot_general(b123, oh12, (((1,), (0,)), ((), ())),
                            preferred_element_type=jnp.float32)
    e12 = (g[:_D] + g[_D:2 * _D]) + g[2 * _D:]

    xx = jnp.concatenate([x, x], axis=1)
    df = e12 - xx
    scr_ref[...] = df * df

    def dstep(d, acc):
        return acc + scr_ref[pl.ds(d, 1), :]

    acc = jax.lax.fori_loop(0, _D, dstep, jnp.zeros((1, 2 * _W), jnp.float32))
    dist = jnp.sqrt(acc)
    d1, d2 = dist[:, :_W], dist[:, _W:]
    a1, a2 = acc[:, :_W], acc[:, _W:]

    win2 = (d2 < d1) | ((d2 == d1) & (c2 < c1))
    idx = jnp.where(win2, c2, c1)
    lsq = jnp.where(win2, a2, a1)

    onehot = (kio == idx).astype(jnp.float32)
    ew = jnp.where(win2, e12[:, _W:], e12[:, :_W])
    qst = x + (ew - x)
    for i in range(_BPS):
        enc_ref[i] = onehot[:, i * _HW:(i + 1) * _HW]
        q_ref[i] = qst[:, i * _HW:(i + 1) * _HW]
    contrib = jnp.sum(lsq).reshape(1, 1)

    @pl.when(b == 0)
    def _():
        cnt_ref[...] = onehot
        loss_ref[...] = contrib

    @pl.when(b > 0)
    def _():
        cnt_ref[...] += onehot
        loss_ref[...] += contrib

    @pl.when(b == _NSTEP - 1)
    def _():
        counts = jnp.sum(cnt_ref[...], axis=1, keepdims=True)
        p = counts * (1.0 / (_B * _HW))
        ent = -jnp.sum(p * jnp.log(p + 1e-10))
        perp_ref[...] = jnp.exp(ent).reshape(1, 1) * (1.0 / _K)
        loss_ref[...] = loss_ref[...] * (1.0 / (_B * _D * _HW))


def kernel(input, embedding):
    B, D, H, W = input.shape
    K = embedding.shape[1]
    x3 = input.reshape(B, D, H * W)
    Em = embedding[:, :, 0]
    enc, q, loss, perp = pl.pallas_call(
        _vq_body,
        grid=(_NSTEP,),
        in_specs=[
            pl.BlockSpec((_BPS, _D, _HW), lambda b: (b, 0, 0)),
            pl.BlockSpec((_D, _K), lambda b: (0, 0)),
        ],
        out_specs=[
            pl.BlockSpec((_BPS, _K, _HW), lambda b: (b, 0, 0)),
            pl.BlockSpec((_BPS, _D, _HW), lambda b: (b, 0, 0)),
            pl.BlockSpec((1, 1), lambda b: (0, 0)),
            pl.BlockSpec((1, 1), lambda b: (0, 0)),
        ],
        out_shape=[
            jax.ShapeDtypeStruct((_B, _K, _HW), jnp.float32),
            jax.ShapeDtypeStruct((_B, _D, _HW), jnp.float32),
            jax.ShapeDtypeStruct((1, 1), jnp.float32),
            jax.ShapeDtypeStruct((1, 1), jnp.float32),
        ],
        scratch_shapes=[
            pltpu.VMEM((_K, _W), jnp.float32),
            pltpu.VMEM((_D, 2 * _W), jnp.float32),
        ],
    )(x3, Em)
    quantized_out = q.reshape(B, D, H, W)
    encoding_out = enc.reshape(B, K, H, W)
    commitment_loss = loss[0, 0]
    perplexity = perp.reshape(1)
    return (quantized_out, encoding_out, commitment_loss, perplexity)


# speedup vs baseline: 1.0498x; 1.0265x over previous
import jax
import jax.numpy as jnp
from jax.experimental import pallas as pl
from jax.experimental.pallas import tpu as pltpu

_B, _D, _K, _HW = 8, 64, 512, 256
_BPS = 8
_W = _BPS * _HW
_NSTEP = _B // _BPS


def _vq_body(x_ref, et_ref, enc_ref, q_ref, loss_ref, perp_ref, cnt_ref,
             scr_ref):
    b = pl.program_id(0)
    x = jnp.concatenate([x_ref[i] for i in range(_BPS)], axis=1)
    Et = et_ref[...]

    b1 = Et.astype(jnp.bfloat16)
    r1 = Et - b1.astype(jnp.float32)
    b2 = r1.astype(jnp.bfloat16)
    r2 = r1 - b2.astype(jnp.float32)
    b3 = r2.astype(jnp.bfloat16)
    x1 = x.astype(jnp.bfloat16)
    x2 = (x - x1.astype(jnp.float32)).astype(jnp.bfloat16)

    def mm(lhs, rhs):
        return jax.lax.dot_general(lhs, rhs, (((1,), (0,)), ((), ())),
                                   preferred_element_type=jnp.float32)

    dots = mm(b1, x1) + (mm(b1, x2) + mm(b2, x1))
    e2 = jnp.sum(Et * Et, axis=1, keepdims=True)
    s = e2 - 2.0 * dots

    kio = jax.lax.broadcasted_iota(jnp.int32, (_K, _W), 0)
    m1 = jnp.min(s, axis=0, keepdims=True)
    eq1 = s == m1
    c1 = jnp.min(jnp.where(eq1, kio, _K), axis=0, keepdims=True)
    s2 = jnp.where(eq1, jnp.inf, s)
    m2 = jnp.min(s2, axis=0, keepdims=True)
    c2 = jnp.min(jnp.where(s2 == m2, kio, _K), axis=0, keepdims=True)

    b123 = jnp.concatenate([b1, b2, b3], axis=1)
    oh12 = jnp.concatenate([(kio == c1), (kio == c2)],
                           axis=1).astype(jnp.bfloat16)
    g = jax.lax.dot_general(b123, oh12, (((0,), (0,)), ((), ())),
                            preferred_element_type=jnp.float32)
    e12 = (g[:_D] + g[_D:2 * _D]) + g[2 * _D:]

    xx = jnp.concatenate([x, x], axis=1)
    df = e12 - xx
    scr_ref[...] = df * df

    def dstep(d, acc):
        return acc + scr_ref[pl.ds(d, 1), :]

    acc = jax.lax.fori_loop(0, _D, dstep, jnp.zeros((1, 2 * _W), jnp.float32))
    dist = jnp.sqrt(acc)
    d1, d2 = dist[:, :_W], dist[:, _W:]
    a1, a2 = acc[:, :_W], acc[:, _W:]

    win2 = (d2 < d1) | ((d2 == d1) & (c2 < c1))
    idx = jnp.where(win2, c2, c1)
    lsq = jnp.where(win2, a2, a1)

    onehot = (kio == idx).astype(jnp.float32)
    ew = jnp.where(win2, e12[:, _W:], e12[:, :_W])
    qst = x + (ew - x)
    for i in range(_BPS):
        enc_ref[i] = onehot[:, i * _HW:(i + 1) * _HW]
        q_ref[i] = qst[:, i * _HW:(i + 1) * _HW]
    contrib = jnp.sum(lsq).reshape(1, 1)

    @pl.when(b == 0)
    def _():
        cnt_ref[...] = onehot
        loss_ref[...] = contrib

    @pl.when(b > 0)
    def _():
        cnt_ref[...] += onehot
        loss_ref[...] += contrib

    @pl.when(b == _NSTEP - 1)
    def _():
        counts = jnp.sum(cnt_ref[...], axis=1, keepdims=True)
        p = counts * (1.0 / (_B * _HW))
        ent = -jnp.sum(p * jnp.log(p + 1e-10))
        perp_ref[...] = jnp.exp(ent).reshape(1, 1) * (1.0 / _K)
        loss_ref[...] = loss_ref[...] * (1.0 / (_B * _D * _HW))


def kernel(input, embedding):
    B, D, H, W = input.shape
    K = embedding.shape[1]
    x3 = input.reshape(B, D, H * W)
    Et = jnp.transpose(embedding[:, :, 0])
    enc, q, loss, perp = pl.pallas_call(
        _vq_body,
        grid=(_NSTEP,),
        in_specs=[
            pl.BlockSpec((_BPS, _D, _HW), lambda b: (b, 0, 0)),
            pl.BlockSpec((_K, _D), lambda b: (0, 0)),
        ],
        out_specs=[
            pl.BlockSpec((_BPS, _K, _HW), lambda b: (b, 0, 0)),
            pl.BlockSpec((_BPS, _D, _HW), lambda b: (b, 0, 0)),
            pl.BlockSpec((1, 1), lambda b: (0, 0)),
            pl.BlockSpec((1, 1), lambda b: (0, 0)),
        ],
        out_shape=[
            jax.ShapeDtypeStruct((_B, _K, _HW), jnp.float32),
            jax.ShapeDtypeStruct((_B, _D, _HW), jnp.float32),
            jax.ShapeDtypeStruct((1, 1), jnp.float32),
            jax.ShapeDtypeStruct((1, 1), jnp.float32),
        ],
        scratch_shapes=[
            pltpu.VMEM((_K, _W), jnp.float32),
            pltpu.VMEM((_D, 2 * _W), jnp.float32),
        ],
    )(x3, Et)
    quantized_out = q.reshape(B, D, H, W)
    encoding_out = enc.reshape(B, K, H, W)
    commitment_loss = loss[0, 0]
    perplexity = perp.reshape(1)
    return (quantized_out, encoding_out, commitment_loss, perplexity)


# speedup vs baseline: 1.1026x; 1.0502x over previous
import jax
import jax.numpy as jnp
from jax.experimental import pallas as pl
from jax.experimental.pallas import tpu as pltpu

_B, _D, _K, _HW = 8, 64, 512, 256
_BPS = 8
_W = _BPS * _HW
_NSTEP = _B // _BPS


def _vq_body(x_ref, et_ref, enc_ref, q_ref, loss_ref, perp_ref, scr_ref):
    x = jnp.concatenate([x_ref[i] for i in range(_BPS)], axis=1)
    Et = et_ref[...]

    b1 = Et.astype(jnp.bfloat16)
    r1 = Et - b1.astype(jnp.float32)
    b2 = r1.astype(jnp.bfloat16)
    r2 = r1 - b2.astype(jnp.float32)
    b3 = r2.astype(jnp.bfloat16)
    x1 = x.astype(jnp.bfloat16)
    x2 = (x - x1.astype(jnp.float32)).astype(jnp.bfloat16)

    lhs3 = jnp.concatenate([b1, b1, b2], axis=1)
    rhs3 = jnp.concatenate([x1, x2, x1], axis=0)
    dots = jax.lax.dot_general(lhs3, rhs3, (((1,), (0,)), ((), ())),
                               preferred_element_type=jnp.float32)
    e2 = jnp.sum(Et * Et, axis=1, keepdims=True)
    s = e2 - 2.0 * dots

    kio = jax.lax.broadcasted_iota(jnp.int32, (_K, _W), 0)
    m1 = jnp.min(s, axis=0, keepdims=True)
    eq1 = s == m1
    c1 = jnp.min(jnp.where(eq1, kio, _K), axis=0, keepdims=True)
    s2 = jnp.where(eq1, jnp.inf, s)
    m2 = jnp.min(s2, axis=0, keepdims=True)
    c2 = jnp.min(jnp.where(s2 == m2, kio, _K), axis=0, keepdims=True)

    b123 = jnp.concatenate([b1, b2, b3], axis=1)
    oh12 = jnp.concatenate([(kio == c1), (kio == c2)],
                           axis=1).astype(jnp.bfloat16)
    g = jax.lax.dot_general(b123, oh12, (((0,), (0,)), ((), ())),
                            preferred_element_type=jnp.float32)
    e12 = (g[:_D] + g[_D:2 * _D]) + g[2 * _D:]

    xx = jnp.concatenate([x, x], axis=1)
    df = e12 - xx
    scr_ref[...] = df * df

    def dstep(d, acc):
        return acc + scr_ref[pl.ds(d, 1), :]

    acc = jax.lax.fori_loop(0, _D, dstep, jnp.zeros((1, 2 * _W), jnp.float32))
    dist = jnp.sqrt(acc)
    d1, d2 = dist[:, :_W], dist[:, _W:]
    a1, a2 = acc[:, :_W], acc[:, _W:]

    win2 = (d2 < d1) | ((d2 == d1) & (c2 < c1))
    idx = jnp.where(win2, c2, c1)
    lsq = jnp.where(win2, a2, a1)

    onehot = (kio == idx).astype(jnp.float32)
    ew = jnp.where(win2, e12[:, _W:], e12[:, :_W])
    qst = x + (ew - x)
    for i in range(_BPS):
        enc_ref[i] = onehot[:, i * _HW:(i + 1) * _HW]
        q_ref[i] = qst[:, i * _HW:(i + 1) * _HW]
    loss_ref[...] = jnp.sum(lsq).reshape(1, 1) * (1.0 / (_B * _D * _HW))
    counts = jnp.sum(onehot, axis=1, keepdims=True)
    p = counts * (1.0 / (_B * _HW))
    ent = -jnp.sum(p * jnp.log(p + 1e-10))
    perp_ref[...] = jnp.exp(ent).reshape(1, 1) * (1.0 / _K)


def kernel(input, embedding):
    B, D, H, W = input.shape
    K = embedding.shape[1]
    x3 = input.reshape(B, D, H * W)
    Et = jnp.transpose(embedding[:, :, 0])
    enc, q, loss, perp = pl.pallas_call(
        _vq_body,
        grid=(_NSTEP,),
        in_specs=[
            pl.BlockSpec((_BPS, _D, _HW), lambda b: (b, 0, 0)),
            pl.BlockSpec((_K, _D), lambda b: (0, 0)),
        ],
        out_specs=[
            pl.BlockSpec((_BPS, _K, _HW), lambda b: (b, 0, 0)),
            pl.BlockSpec((_BPS, _D, _HW), lambda b: (b, 0, 0)),
            pl.BlockSpec((1, 1), lambda b: (0, 0)),
            pl.BlockSpec((1, 1), lambda b: (0, 0)),
        ],
        out_shape=[
            jax.ShapeDtypeStruct((_B, _K, _HW), jnp.float32),
            jax.ShapeDtypeStruct((_B, _D, _HW), jnp.float32),
            jax.ShapeDtypeStruct((1, 1), jnp.float32),
            jax.ShapeDtypeStruct((1, 1), jnp.float32),
        ],
        scratch_shapes=[
            pltpu.VMEM((_D, 2 * _W), jnp.float32),
        ],
    )(x3, Et)
    quantized_out = q.reshape(B, D, H, W)
    encoding_out = enc.reshape(B, K, H, W)
    commitment_loss = loss[0, 0]
    perplexity = perp.reshape(1)
    return (quantized_out, encoding_out, commitment_loss, perplexity)
